# Initial kernel scaffold; baseline (speedup 1.0000x reference)
import jax
import jax.numpy as jnp
from jax.experimental import pallas as pl


def kernel(pred, target, flow, c_flow):
    raise NotImplementedError("write your pallas kernel here")


# speedup vs baseline: 31.1669x; 31.1669x over previous
import functools

import jax
import jax.numpy as jnp
from jax import lax
from jax.experimental import pallas as pl
from jax.experimental.pallas import tpu as pltpu
from jax.experimental.pallas import tpu_sc as plsc

N_NODES_PAD = 100096
N_UPD = 2 * 6400000
NC, NS = 2, 16
NW = NC * NS
PER_W = N_UPD // NW
CHUNK = 8000
N_CHUNK = PER_W // CHUNK
SLICE = N_NODES_PAD // NS
NVREG = SLICE // 16


def _body(idx_hbm, val_hbm, out_hbm, idx_v, val_v, red_v, acc16_v, acc_sh):
    cid = lax.axis_index("c")
    sid = lax.axis_index("s")
    wid = sid * NC + cid

    def _zero(i, _):
        red_v[pl.ds(i * 16, 16)] = jnp.zeros((16,), jnp.float32)
        return 0

    lax.fori_loop(0, NVREG, _zero, 0)
    pltpu.sync_copy(red_v, acc_sh.at[pl.ds(sid * SLICE, SLICE)])
    plsc.subcore_barrier()

    base = wid * PER_W
    for k in range(N_CHUNK):
        off = base + k * CHUNK
        pltpu.sync_copy(idx_hbm.at[pl.ds(off, CHUNK)], idx_v)
        pltpu.sync_copy(val_hbm.at[pl.ds(off, CHUNK)], val_v)
        pltpu.sync_copy(val_v, acc_sh.at[idx_v], add=True)
    plsc.subcore_barrier()

    pltpu.sync_copy(acc_sh.at[pl.ds(sid * SLICE, SLICE)], red_v)

    def _red(j, a):
        return a + red_v[pl.ds(j * 16, 16)]

    acc16_v[...] = lax.fori_loop(0, NVREG, _red, jnp.zeros((16,), jnp.float32))
    pltpu.sync_copy(acc16_v, out_hbm.at[wid])


@jax.jit
def _run(idx, val):
    mesh = plsc.VectorSubcoreMesh(core_axis_name="c", subcore_axis_name="s")
    f = functools.partial(
        pl.kernel,
        mesh=mesh,
        out_type=jax.ShapeDtypeStruct((NW, 16), jnp.float32),
        scratch_types=[
            pltpu.VMEM((CHUNK,), jnp.int32),
            pltpu.VMEM((CHUNK,), jnp.float32),
            pltpu.VMEM((SLICE,), jnp.float32),
            pltpu.VMEM((16,), jnp.float32),
            pltpu.VMEM_SHARED((N_NODES_PAD,), jnp.float32),
        ],
    )(_body)
    return f(idx, val)


def kernel(pred, target, flow, c_flow):
    src = flow[:, 0].astype(jnp.int32)
    dst = flow[:, 1].astype(jnp.int32)
    vals = flow[:, 2]
    idx = jnp.concatenate([src, dst])
    val = jnp.concatenate([vals, -vals])
    partials = _run(idx, val)
    return jnp.sum(partials)
